# baseline (speedup 1.0000x reference)
import functools

import jax
import jax.numpy as jnp
from jax import lax
from jax.experimental import pallas as pl
from jax.experimental.pallas import tpu as pltpu
from jax.experimental.pallas import tpu_sc as plsc

D = 768
E = 16
N = 16
F = 4 * D
C = 1536
NC = F // C

SC_CORES = 2
SC_SUBCORES = 16
NW = SC_CORES * SC_SUBCORES
RPT = F // NW
CH = 48


def _gate_z_body(x_ref, wg_ref, wfc_ref, z_ref, g_ref):
    i = pl.program_id(0)
    c = pl.program_id(1)

    @pl.when((i == 0) & (c == 0))
    def _gating():
        xf = x_ref[:]
        logits = jnp.dot(xf, wg_ref[:],
                         preferred_element_type=jnp.float32)
        m = jnp.max(logits, axis=-1, keepdims=True)
        ex = jnp.exp(logits - m)
        gates = ex / jnp.sum(ex, axis=-1, keepdims=True)
        gk = gates[:, :, None]
        gm = gates[:, None, :]
        iota_k = jax.lax.broadcasted_iota(jnp.int32, (N, E, E), 1)
        iota_m = jax.lax.broadcasted_iota(jnp.int32, (N, E, E), 2)
        before = (gm > gk) | ((gm == gk) & (iota_m < iota_k))
        rank = jnp.sum(before.astype(jnp.int32), axis=2)
        onehot = (rank[:, :, None]
                  == jax.lax.broadcasted_iota(jnp.int32, (N, E, E), 2))
        srt = jnp.sum(gates[:, :, None] * onehot.astype(jnp.float32), axis=1)
        g_ref[:] = srt / jnp.sum(srt, axis=-1, keepdims=True)

    h = jnp.dot(x_ref[:], wfc_ref[0], preferred_element_type=jnp.float32)
    a = 0.5 * h * (1.0 + jax.lax.erf(h * 0.7071067811865476))
    grow = g_ref[pl.ds(i, 1), :]
    z_ref[0] = jnp.dot(grow, a, preferred_element_type=jnp.float32)


def _sc_proj_body(z_hbm, wp_hbm, out_hbm, wbuf, zbuf, obuf, rcnt):
    cid = lax.axis_index("c")
    sid = lax.axis_index("s")
    wid = sid * SC_CORES + cid
    base = wid * RPT

    def zero_body(t, carry):
        obuf[pl.ds(t * 16, 16)] = jnp.zeros((16,), jnp.float32)
        return carry
    lax.fori_loop(0, E * D // 16, zero_body, 0)

    def expert_body(i, carry):
        pltpu.sync_copy(z_hbm.at[pl.ds(i * F + base, RPT)], zbuf)
        rcnt[:] = jnp.zeros((16,), jnp.int32)

        def chunk_body(ch, carry2):
            pltpu.sync_copy(
                wp_hbm.at[pl.ds(i * F + base + ch * CH, CH), :], wbuf)

            def group_body(g, carry3):
                zg = zbuf[pl.ds(ch * CH + g * 16, 16)]
                rcnt[:] = jnp.zeros((16,), jnp.int32)

                def row_body(r2, carry4):
                    idx = rcnt[:]
                    zb = lax.gather(
                        zg, idx[:, None],
                        lax.GatherDimensionNumbers(
                            offset_dims=(), collapsed_slice_dims=(0,),
                            start_index_map=(0,)),
                        slice_sizes=(1,),
                        mode=lax.GatherScatterMode.PROMISE_IN_BOUNDS)
                    rcnt[:] = idx + jnp.full((16,), 1, jnp.int32)
                    r = g * 16 + r2

                    def j_body(j, carry5):
                        v = wbuf[r, pl.ds(j * 16, 16)]
                        plsc.addupdate(
                            obuf.at[pl.ds(i * D + j * 16, 16)], zb * v)
                        return carry5
                    return lax.fori_loop(0, D // 16, j_body, carry4)
                return lax.fori_loop(0, 16, row_body, carry3)
            return lax.fori_loop(0, CH // 16, group_body, carry2)
        return lax.fori_loop(0, F // (NW * CH), chunk_body, carry)
    lax.fori_loop(0, E, expert_body, 0)

    pltpu.sync_copy(obuf, out_hbm.at[pl.ds(wid * E * D, E * D)])


def _reduce_body(p_ref, out_ref):
    out_ref[:] = jnp.sum(p_ref[:], axis=0, keepdims=True)


def kernel(x, Wg, Wfc, Wproj):
    orig_shape = x.shape
    xf = x.reshape(-1, D)

    z = pl.pallas_call(
        _gate_z_body,
        grid=(E, NC),
        in_specs=[
            pl.BlockSpec((N, D), lambda i, c: (0, 0)),
            pl.BlockSpec((D, E), lambda i, c: (0, 0)),
            pl.BlockSpec((1, D, C), lambda i, c: (i, 0, c)),
        ],
        out_specs=pl.BlockSpec((1, 1, C), lambda i, c: (i, 0, c)),
        out_shape=jax.ShapeDtypeStruct((E, 1, F), jnp.float32),
        scratch_shapes=[pltpu.VMEM((N, E), jnp.float32)],
        compiler_params=pltpu.CompilerParams(
            dimension_semantics=("arbitrary", "arbitrary"),
        ),
    )(xf, Wg, Wfc)
    z2 = z.reshape(E * F)

    mesh = plsc.VectorSubcoreMesh(
        core_axis_name="c", subcore_axis_name="s",
        num_cores=SC_CORES, num_subcores=SC_SUBCORES)
    partials = pl.kernel(
        _sc_proj_body,
        out_type=jax.ShapeDtypeStruct((NW * E * D,), jnp.float32),
        mesh=mesh,
        scratch_types=[
            pltpu.VMEM((CH, D), jnp.float32),
            pltpu.VMEM((RPT,), jnp.float32),
            pltpu.VMEM((E * D,), jnp.float32),
            pltpu.VMEM((16,), jnp.int32),
        ],
    )(z2, Wproj.reshape(E * F, D))

    out = pl.pallas_call(
        _reduce_body,
        in_specs=[pl.BlockSpec((NW, N * D), lambda: (0, 0))],
        out_specs=pl.BlockSpec((1, N * D), lambda: (0, 0)),
        out_shape=jax.ShapeDtypeStruct((1, N * D), jnp.float32),
    )(partials.reshape(NW, N * D))
    return out.reshape(orig_shape)


# speedup vs baseline: 5.6596x; 5.6596x over previous
import jax
import jax.numpy as jnp
from jax.experimental import pallas as pl
from jax.experimental.pallas import tpu as pltpu

D = 768
E = 16
N = 16
F = 4 * D
C = 1536
NC = F // C


def _gate_body(x_ref, wg_ref, g_ref):
    xf = x_ref[:]
    logits = jnp.dot(xf, wg_ref[:],
                     preferred_element_type=jnp.float32)
    m = jnp.max(logits, axis=-1, keepdims=True)
    ex = jnp.exp(logits - m)
    gates = ex / jnp.sum(ex, axis=-1, keepdims=True)
    gk = gates[:, :, None]
    gm = gates[:, None, :]
    iota_k = jax.lax.broadcasted_iota(jnp.int32, (N, E, E), 1)
    iota_m = jax.lax.broadcasted_iota(jnp.int32, (N, E, E), 2)
    before = (gm > gk) | ((gm == gk) & (iota_m < iota_k))
    rank = jnp.sum(before.astype(jnp.int32), axis=2)
    onehot = (rank[:, :, None]
              == jax.lax.broadcasted_iota(jnp.int32, (N, E, E), 2))
    srt = jnp.sum(gates[:, :, None] * onehot.astype(jnp.float32), axis=1)
    g_ref[:] = srt / jnp.sum(srt, axis=-1, keepdims=True)


def _moe_body(g_ref, x_ref, wfc_ref, wproj_ref, out_ref):
    i = pl.program_id(0)
    c = pl.program_id(1)

    h = jnp.dot(x_ref[:], wfc_ref[0], preferred_element_type=jnp.float32)
    a = 0.5 * h * (1.0 + jax.lax.erf(h * 0.7071067811865476))
    grow = g_ref[pl.ds(i, 1), :]
    z = jnp.dot(grow, a, preferred_element_type=jnp.float32)
    part = jnp.dot(z, wproj_ref[0], preferred_element_type=jnp.float32)

    @pl.when(c == 0)
    def _init():
        out_ref[0] = part

    @pl.when(c != 0)
    def _acc():
        out_ref[0] += part


def kernel(x, Wg, Wfc, Wproj):
    orig_shape = x.shape
    xf = x.reshape(-1, D)
    g = pl.pallas_call(
        _gate_body,
        out_shape=jax.ShapeDtypeStruct((N, E), jnp.float32),
    )(xf, Wg)
    out = pl.pallas_call(
        _moe_body,
        grid=(E, NC),
        in_specs=[
            pl.BlockSpec((N, E), lambda i, c: (0, 0)),
            pl.BlockSpec((N, D), lambda i, c: (0, 0)),
            pl.BlockSpec((1, D, C), lambda i, c: (i, 0, c)),
            pl.BlockSpec((1, C, D), lambda i, c: (i, c, 0)),
        ],
        out_specs=pl.BlockSpec((1, 1, D), lambda i, c: (i, 0, 0)),
        out_shape=jax.ShapeDtypeStruct((E, 1, D), jnp.float32),
        compiler_params=pltpu.CompilerParams(
            dimension_semantics=("parallel", "arbitrary"),
        ),
    )(g, xf, Wfc, Wproj)
    return out.reshape(orig_shape)


# speedup vs baseline: 5.7569x; 1.0172x over previous
import jax
import jax.numpy as jnp
from jax.experimental import pallas as pl
from jax.experimental.pallas import tpu as pltpu

D = 768
E = 16
N = 16
F = 4 * D
C = 1536
NC = F // C


def _moe_body(x_ref, wg_ref, wfc_ref, wproj_ref, out_ref, g_ref):
    i = pl.program_id(0)
    c = pl.program_id(1)

    @pl.when((i == 0) & (c == 0))
    def _gating():
        xf = x_ref[:]
        logits = jnp.dot(xf, wg_ref[:],
                         preferred_element_type=jnp.float32)
        m = jnp.max(logits, axis=-1, keepdims=True)
        ex = jnp.exp(logits - m)
        gates = ex / jnp.sum(ex, axis=-1, keepdims=True)
        gk = gates[:, :, None]
        gm = gates[:, None, :]
        iota_k = jax.lax.broadcasted_iota(jnp.int32, (N, E, E), 1)
        iota_m = jax.lax.broadcasted_iota(jnp.int32, (N, E, E), 2)
        before = (gm > gk) | ((gm == gk) & (iota_m < iota_k))
        rank = jnp.sum(before.astype(jnp.int32), axis=2)
        onehot = (rank[:, :, None]
                  == jax.lax.broadcasted_iota(jnp.int32, (N, E, E), 2))
        srt = jnp.sum(gates[:, :, None] * onehot.astype(jnp.float32), axis=1)
        srt = srt / jnp.sum(srt, axis=-1, keepdims=True)
        g_ref[:] = srt

    h = jnp.dot(x_ref[:], wfc_ref[0], preferred_element_type=jnp.float32)
    a = 0.5 * h * (1.0 + jax.lax.erf(h * 0.7071067811865476))
    grow = g_ref[pl.ds(i, 1), :]
    z = jnp.dot(grow, a, preferred_element_type=jnp.float32)
    part = jnp.dot(z, wproj_ref[0], preferred_element_type=jnp.float32)

    @pl.when(c == 0)
    def _init():
        out_ref[0] = part

    @pl.when(c != 0)
    def _acc():
        out_ref[0] += part


def kernel(x, Wg, Wfc, Wproj):
    orig_shape = x.shape
    xf = x.reshape(-1, D)
    out = pl.pallas_call(
        _moe_body,
        grid=(E, NC),
        in_specs=[
            pl.BlockSpec((N, D), lambda i, c: (0, 0)),
            pl.BlockSpec((D, E), lambda i, c: (0, 0)),
            pl.BlockSpec((1, D, C), lambda i, c: (i, 0, c)),
            pl.BlockSpec((1, C, D), lambda i, c: (i, c, 0)),
        ],
        out_specs=pl.BlockSpec((1, 1, D), lambda i, c: (i, 0, 0)),
        out_shape=jax.ShapeDtypeStruct((E, 1, D), jnp.float32),
        scratch_shapes=[pltpu.VMEM((N, E), jnp.float32)],
        compiler_params=pltpu.CompilerParams(
            dimension_semantics=("arbitrary", "arbitrary"),
        ),
    )(xf, Wg, Wfc, Wproj)
    return out.reshape(orig_shape)
